# baseline (speedup 1.0000x reference)
import functools

import jax
import jax.numpy as jnp
from jax import lax
from jax.experimental import pallas as pl
from jax.experimental.pallas import tpu as pltpu
from jax.experimental.pallas import tpu_sc as plsc


def _transform_body(x_ref, f_ref, o_ref):
    x = x_ref[0]
    sub = x.shape[1] // 8
    acc = jnp.zeros((sub, 128), jnp.float32)
    for s in range(8):
        xs = x[:, s * sub:(s + 1) * sub]
        acc = acc + lax.dot_general(
            xs, f_ref[s], (((0,), (0,)), ((), ())),
            preferred_element_type=jnp.float32,
        )
    o_ref[0] = acc


def _tc_transform(tables_t, w2, vb=8192):
    j_n, d, v = tables_t.shape
    assert d == 16
    gv = (v + vb - 1) // vb
    sub = vb // 8
    f = jnp.zeros((8, d, 128), jnp.float32)
    f = f.at[jnp.arange(8)[:, None, None], jnp.arange(d)[None, :, None],
             (16 * jnp.arange(8))[:, None, None] + jnp.arange(d)[None, None, :]
             ].set(jnp.broadcast_to(w2[None], (8, d, d)))
    return pl.pallas_call(
        _transform_body,
        grid=(j_n, gv),
        in_specs=[
            pl.BlockSpec((1, d, vb), lambda j, g: (j, 0, g)),
            pl.BlockSpec((8, d, 128), lambda j, g: (0, 0, 0)),
        ],
        out_specs=pl.BlockSpec((1, sub, 128), lambda j, g: (j, g, 0)),
        out_shape=jax.ShapeDtypeStruct((j_n, gv * sub, 128), jnp.float32),
    )(tables_t, f)


def _sc_gather(tables_flat, idx2d):
    R, L = idx2d.shape
    D = tables_flat.shape[1]
    info = plsc.get_sparse_core_info()
    nc, ns = info.num_cores, info.num_subcores
    nw = nc * ns
    assert R % nw == 0, (R, nw)
    rows_per_w = R // nw
    assert rows_per_w % 8 == 0, rows_per_w
    k = next(x for x in (16, 8) if rows_per_w % x == 0)
    n_chunks = rows_per_w // k

    def body(tbl, idx, out, idx_v, rows_v, sem):
        wid = lax.axis_index("s") * nc + lax.axis_index("c")
        base = wid * rows_per_w

        def chunk(c, carry):
            r0 = base + c * k
            pltpu.sync_copy(idx.at[pl.ds(r0, k)], idx_v)
            handles = [
                pltpu.async_copy(tbl.at[idx_v.at[i]], rows_v.at[i], sem)
                for i in range(k)
            ]
            for h in handles:
                h.wait()
            pltpu.sync_copy(rows_v, out.at[pl.ds(r0, k)])
            return carry

        lax.fori_loop(0, n_chunks, chunk, 0)

    f = pl.kernel(
        body,
        mesh=plsc.VectorSubcoreMesh(core_axis_name="c", subcore_axis_name="s"),
        compiler_params=pltpu.CompilerParams(use_tc_tiling_on_sc=False),
        out_type=jax.ShapeDtypeStruct((R, L, D), jnp.float32),
        scratch_types=[
            pltpu.VMEM((k, L), jnp.int32),
            pltpu.VMEM((k, L, D), jnp.float32),
            pltpu.SemaphoreType.DMA,
        ],
    )
    return f(tables_flat, idx2d)


def _tc_body(x_ref, v_ref, mn_ref, mc_ref, c_ref, o_ref):
    o_ref[...] = (
        jnp.dot(x_ref[...], mn_ref[...], preferred_element_type=jnp.float32)
        + jnp.dot(v_ref[...], mc_ref[...], preferred_element_type=jnp.float32)
        + c_ref[...][None, :]
    )


def _tc_fuse(x_num, val2d, m_num, m_cat, c_row, block_b=1024):
    bsz = x_num.shape[0]
    n_num = x_num.shape[1]
    wc = val2d.shape[1]
    wo = c_row.shape[0]
    assert bsz % block_b == 0
    return pl.pallas_call(
        _tc_body,
        grid=(bsz // block_b,),
        in_specs=[
            pl.BlockSpec((block_b, n_num), lambda i: (i, 0)),
            pl.BlockSpec((block_b, wc), lambda i: (i, 0)),
            pl.BlockSpec((n_num, wo), lambda i: (0, 0)),
            pl.BlockSpec((wc, wo), lambda i: (0, 0)),
            pl.BlockSpec((wo,), lambda i: (0,)),
        ],
        out_specs=pl.BlockSpec((block_b, wo), lambda i: (i, 0)),
        out_shape=jax.ShapeDtypeStruct((bsz, wo), jnp.float32),
    )(x_num, val2d, m_num, m_cat, c_row)


def kernel(X_num, X_cat, feature_emb, W_num, b_num, cat_tables, W_proj, b_proj, cls_token):
    bsz, n_num = X_num.shape
    n_cat = X_cat.shape[1]
    card = cat_tables.shape[1]
    d = feature_emb.shape[1]
    n_tok = 1 + n_num + n_cat
    wo = n_tok * d

    w1 = W_proj[:d]
    w2 = W_proj[d:]
    v_vec = W_num[0] @ w2
    c_num = feature_emb[:n_num] @ w1 + b_proj + b_num @ w2
    c_cat = feature_emb[n_num:] @ w1 + b_proj
    c_row = jnp.concatenate(
        [cls_token.reshape(d), c_num.reshape(-1), c_cat.reshape(-1)]
    )
    m_num = jnp.einsum("ij,k->ijk", jnp.eye(n_num, dtype=jnp.float32), v_vec)
    m_num = jnp.pad(m_num.reshape(n_num, n_num * d), ((0, 0), (d, n_cat * d)))
    m_cat = jnp.pad(jnp.eye(n_cat * d, dtype=jnp.float32), ((0, 0), ((1 + n_num) * d, 0)))

    vb = 8192
    sub = vb // 8
    gv = (card + vb - 1) // vb
    tables_t = jnp.transpose(cat_tables, (0, 2, 1))
    t2 = _tc_transform(tables_t, w2, vb=vb)
    t2_flat = t2.reshape(n_cat * gv * sub * 8, d)

    offs = (jnp.arange(n_cat, dtype=jnp.int32) * gv)[None, :]
    g = X_cat // vb
    s = (X_cat % vb) // sub
    r = X_cat % sub
    flat_idx = ((((offs + g) * sub + r) * 8) + s).reshape(-1)
    assert flat_idx.shape[0] % 128 == 0
    idx2d = flat_idx.reshape(-1, 128)

    rows = _sc_gather(t2_flat, idx2d)
    val2d = rows.reshape(bsz, n_cat * d)

    out2d = _tc_fuse(X_num, val2d, m_num, m_cat, c_row)
    return out2d.reshape(bsz, n_tok, d)


# speedup vs baseline: 1.0227x; 1.0227x over previous
import functools

import jax
import jax.numpy as jnp
from jax import lax
from jax.experimental import pallas as pl
from jax.experimental.pallas import tpu as pltpu
from jax.experimental.pallas import tpu_sc as plsc


def _transform_body(x_ref, f_ref, o_ref):
    x = x_ref[0]
    sub = x.shape[1] // 8
    parts = [
        lax.dot_general(
            x[:, s * sub:(s + 1) * sub], f_ref[s],
            (((0,), (0,)), ((), ())),
            preferred_element_type=jnp.float32,
        )
        for s in range(8)
    ]
    while len(parts) > 1:
        parts = [a + b for a, b in zip(parts[::2], parts[1::2])]
    o_ref[0] = parts[0]


def _tc_transform(tables_t, f, j0, j_n, vb=8192):
    _, d, v = tables_t.shape
    assert d == 16
    gv = (v + vb - 1) // vb
    sub = vb // 8
    return pl.pallas_call(
        _transform_body,
        grid=(j_n, gv),
        in_specs=[
            pl.BlockSpec((1, d, vb), lambda j, g: (j + j0, 0, g)),
            pl.BlockSpec((8, d, 128), lambda j, g: (0, 0, 0)),
        ],
        out_specs=pl.BlockSpec((1, sub, 128), lambda j, g: (j, g, 0)),
        out_shape=jax.ShapeDtypeStruct((j_n, gv * sub, 128), jnp.float32),
    )(tables_t, f)


def _sc_gather(tables_flat, idx2d):
    R, L = idx2d.shape
    D = tables_flat.shape[1]
    info = plsc.get_sparse_core_info()
    nc, ns = info.num_cores, info.num_subcores
    nw = nc * ns
    assert R % nw == 0, (R, nw)
    rows_per_w = R // nw
    assert rows_per_w % 8 == 0, rows_per_w
    k = next(x for x in (16, 8) if rows_per_w % x == 0)
    n_chunks = rows_per_w // k

    def body(tbl, idx, out, idx_v, rows_v, sem):
        wid = lax.axis_index("s") * nc + lax.axis_index("c")
        base = wid * rows_per_w

        def chunk(c, carry):
            r0 = base + c * k
            pltpu.sync_copy(idx.at[pl.ds(r0, k)], idx_v)
            handles = [
                pltpu.async_copy(tbl.at[idx_v.at[i]], rows_v.at[i], sem)
                for i in range(k)
            ]
            for h in handles:
                h.wait()
            pltpu.sync_copy(rows_v, out.at[pl.ds(r0, k)])
            return carry

        lax.fori_loop(0, n_chunks, chunk, 0)

    f = pl.kernel(
        body,
        mesh=plsc.VectorSubcoreMesh(core_axis_name="c", subcore_axis_name="s"),
        compiler_params=pltpu.CompilerParams(use_tc_tiling_on_sc=False),
        out_type=jax.ShapeDtypeStruct((R, L, D), jnp.float32),
        scratch_types=[
            pltpu.VMEM((k, L), jnp.int32),
            pltpu.VMEM((k, L, D), jnp.float32),
            pltpu.SemaphoreType.DMA,
        ],
    )
    return f(tables_flat, idx2d)


def _tc_body(x_ref, v0_ref, v1_ref, mn_ref, m0_ref, m1_ref, c_ref, o_ref):
    o_ref[...] = (
        jnp.dot(x_ref[...], mn_ref[...], preferred_element_type=jnp.float32)
        + jnp.dot(v0_ref[...], m0_ref[...], preferred_element_type=jnp.float32)
        + jnp.dot(v1_ref[...], m1_ref[...], preferred_element_type=jnp.float32)
        + c_ref[...][None, :]
    )


def _tc_fuse(x_num, val0, val1, m_num, m0, m1, c_row, block_b=1024):
    bsz = x_num.shape[0]
    n_num = x_num.shape[1]
    w0 = val0.shape[1]
    w1 = val1.shape[1]
    wo = c_row.shape[0]
    assert bsz % block_b == 0
    return pl.pallas_call(
        _tc_body,
        grid=(bsz // block_b,),
        in_specs=[
            pl.BlockSpec((block_b, n_num), lambda i: (i, 0)),
            pl.BlockSpec((block_b, w0), lambda i: (i, 0)),
            pl.BlockSpec((block_b, w1), lambda i: (i, 0)),
            pl.BlockSpec((n_num, wo), lambda i: (0, 0)),
            pl.BlockSpec((w0, wo), lambda i: (0, 0)),
            pl.BlockSpec((w1, wo), lambda i: (0, 0)),
            pl.BlockSpec((wo,), lambda i: (0,)),
        ],
        out_specs=pl.BlockSpec((block_b, wo), lambda i: (i, 0)),
        out_shape=jax.ShapeDtypeStruct((bsz, wo), jnp.float32),
    )(x_num, val0, val1, m_num, m0, m1, c_row)


def kernel(X_num, X_cat, feature_emb, W_num, b_num, cat_tables, W_proj, b_proj, cls_token):
    bsz, n_num = X_num.shape
    n_cat = X_cat.shape[1]
    card = cat_tables.shape[1]
    d = feature_emb.shape[1]
    n_tok = 1 + n_num + n_cat
    wo = n_tok * d

    w1 = W_proj[:d]
    w2 = W_proj[d:]
    v_vec = W_num[0] @ w2
    c_num = feature_emb[:n_num] @ w1 + b_proj + b_num @ w2
    c_cat = feature_emb[n_num:] @ w1 + b_proj
    c_row = jnp.concatenate(
        [cls_token.reshape(d), c_num.reshape(-1), c_cat.reshape(-1)]
    )
    m_num = jnp.einsum("ij,k->ijk", jnp.eye(n_num, dtype=jnp.float32), v_vec)
    m_num = jnp.pad(m_num.reshape(n_num, n_num * d), ((0, 0), (d, n_cat * d)))

    vb = 8192
    sub = vb // 8
    gv = (card + vb - 1) // vb
    tables_t = jnp.transpose(cat_tables, (0, 2, 1))
    f = jnp.zeros((8, d, 128), jnp.float32)
    f = f.at[jnp.arange(8)[:, None, None], jnp.arange(d)[None, :, None],
             (16 * jnp.arange(8))[:, None, None] + jnp.arange(d)[None, None, :]
             ].set(jnp.broadcast_to(w2[None], (8, d, d)))

    g_all = X_cat // vb
    s_all = (X_cat % vb) // sub
    r_all = X_cat % sub

    splits = [(0, 16), (16, n_cat)]
    vals = []
    for j0, j1 in splits:
        jn = j1 - j0
        t2 = _tc_transform(tables_t, f, j0, jn, vb=vb)
        t2_flat = t2.reshape(jn * gv * sub * 8, d)
        offs = (jnp.arange(jn, dtype=jnp.int32) * gv)[None, :]
        flat_idx = (
            (((offs + g_all[:, j0:j1]) * sub + r_all[:, j0:j1]) * 8)
            + s_all[:, j0:j1]
        ).reshape(-1)
        assert flat_idx.shape[0] % 128 == 0
        idx2d = flat_idx.reshape(-1, 128)
        rows = _sc_gather(t2_flat, idx2d)
        vals.append(rows.reshape(bsz, jn * d))

    w0 = vals[0].shape[1]
    w1c = vals[1].shape[1]
    m0 = jnp.pad(jnp.eye(w0, dtype=jnp.float32),
                 ((0, 0), ((1 + n_num) * d, wo - (1 + n_num) * d - w0)))
    m1 = jnp.pad(jnp.eye(w1c, dtype=jnp.float32),
                 ((0, 0), ((1 + n_num) * d + w0, 0)))

    out2d = _tc_fuse(X_num, vals[0], vals[1], m_num, m0, m1, c_row)
    return out2d.reshape(bsz, n_tok, d)


# speedup vs baseline: 1.1926x; 1.1662x over previous
import functools

import jax
import jax.numpy as jnp
from jax import lax
from jax.experimental import pallas as pl
from jax.experimental.pallas import tpu as pltpu
from jax.experimental.pallas import tpu_sc as plsc


def _transform_body(x_ref, f_ref, o_ref):
    x = x_ref[0].astype(jnp.bfloat16)
    sub = x.shape[1] // 8
    parts = [
        lax.dot_general(
            x[:, s * sub:(s + 1) * sub], f_ref[s],
            (((0,), (0,)), ((), ())),
            preferred_element_type=jnp.float32,
        )
        for s in range(8)
    ]
    while len(parts) > 1:
        parts = [a + b for a, b in zip(parts[::2], parts[1::2])]
    o_ref[0] = parts[0]


def _tc_transform(tables_t, f, j0, j_n, vb=8192):
    _, d, v = tables_t.shape
    assert d == 16
    gv = (v + vb - 1) // vb
    sub = vb // 8
    return pl.pallas_call(
        _transform_body,
        grid=(j_n, gv),
        in_specs=[
            pl.BlockSpec((1, d, vb), lambda j, g: (j + j0, 0, g)),
            pl.BlockSpec((8, d, 128), lambda j, g: (0, 0, 0)),
        ],
        out_specs=pl.BlockSpec((1, sub, 128), lambda j, g: (j, g, 0)),
        out_shape=jax.ShapeDtypeStruct((j_n, gv * sub, 128), jnp.float32),
    )(tables_t, f)


def _sc_gather(tables_flat, idx2d):
    R, L = idx2d.shape
    D = tables_flat.shape[1]
    info = plsc.get_sparse_core_info()
    nc, ns = info.num_cores, info.num_subcores
    nw = nc * ns
    assert R % nw == 0, (R, nw)
    rows_per_w = R // nw
    assert rows_per_w % 8 == 0, rows_per_w
    k = next(x for x in (16, 8) if rows_per_w % x == 0)
    n_chunks = rows_per_w // k

    def body(tbl, idx, out, idx_v, rows_v, sem):
        wid = lax.axis_index("s") * nc + lax.axis_index("c")
        base = wid * rows_per_w

        def chunk(c, carry):
            r0 = base + c * k
            pltpu.sync_copy(idx.at[pl.ds(r0, k)], idx_v)
            handles = [
                pltpu.async_copy(tbl.at[idx_v.at[i]], rows_v.at[i], sem)
                for i in range(k)
            ]
            for h in handles:
                h.wait()
            pltpu.sync_copy(rows_v, out.at[pl.ds(r0, k)])
            return carry

        lax.fori_loop(0, n_chunks, chunk, 0)

    f = pl.kernel(
        body,
        mesh=plsc.VectorSubcoreMesh(core_axis_name="c", subcore_axis_name="s"),
        compiler_params=pltpu.CompilerParams(use_tc_tiling_on_sc=False),
        out_type=jax.ShapeDtypeStruct((R, L, D), jnp.float32),
        scratch_types=[
            pltpu.VMEM((k, L), jnp.int32),
            pltpu.VMEM((k, L, D), jnp.float32),
            pltpu.SemaphoreType.DMA,
        ],
    )
    return f(tables_flat, idx2d)


def _tc_body(x_ref, v0_ref, v1_ref, mn_ref, m0_ref, m1_ref, c_ref, o_ref):
    o_ref[...] = (
        jnp.dot(x_ref[...], mn_ref[...], preferred_element_type=jnp.float32)
        + jnp.dot(v0_ref[...], m0_ref[...], preferred_element_type=jnp.float32)
        + jnp.dot(v1_ref[...], m1_ref[...], preferred_element_type=jnp.float32)
        + c_ref[...][None, :]
    )


def _tc_fuse(x_num, val0, val1, m_num, m0, m1, c_row, block_b=1024):
    bsz = x_num.shape[0]
    n_num = x_num.shape[1]
    w0 = val0.shape[1]
    w1 = val1.shape[1]
    wo = c_row.shape[0]
    assert bsz % block_b == 0
    return pl.pallas_call(
        _tc_body,
        grid=(bsz // block_b,),
        in_specs=[
            pl.BlockSpec((block_b, n_num), lambda i: (i, 0)),
            pl.BlockSpec((block_b, w0), lambda i: (i, 0)),
            pl.BlockSpec((block_b, w1), lambda i: (i, 0)),
            pl.BlockSpec((n_num, wo), lambda i: (0, 0)),
            pl.BlockSpec((w0, wo), lambda i: (0, 0)),
            pl.BlockSpec((w1, wo), lambda i: (0, 0)),
            pl.BlockSpec((wo,), lambda i: (0,)),
        ],
        out_specs=pl.BlockSpec((block_b, wo), lambda i: (i, 0)),
        out_shape=jax.ShapeDtypeStruct((bsz, wo), jnp.float32),
    )(x_num, val0, val1, m_num, m0, m1, c_row)


def kernel(X_num, X_cat, feature_emb, W_num, b_num, cat_tables, W_proj, b_proj, cls_token):
    bsz, n_num = X_num.shape
    n_cat = X_cat.shape[1]
    card = cat_tables.shape[1]
    d = feature_emb.shape[1]
    n_tok = 1 + n_num + n_cat
    wo = n_tok * d

    w1 = W_proj[:d]
    w2 = W_proj[d:]
    v_vec = W_num[0] @ w2
    c_num = feature_emb[:n_num] @ w1 + b_proj + b_num @ w2
    c_cat = feature_emb[n_num:] @ w1 + b_proj
    c_row = jnp.concatenate(
        [cls_token.reshape(d), c_num.reshape(-1), c_cat.reshape(-1)]
    )
    m_num = jnp.einsum("ij,k->ijk", jnp.eye(n_num, dtype=jnp.float32), v_vec)
    m_num = jnp.pad(m_num.reshape(n_num, n_num * d), ((0, 0), (d, n_cat * d)))

    vb = 8192
    sub = vb // 8
    gv = (card + vb - 1) // vb
    tables_t = jnp.transpose(cat_tables, (0, 2, 1))
    f = jnp.zeros((8, d, 128), jnp.float32)
    f = f.at[jnp.arange(8)[:, None, None], jnp.arange(d)[None, :, None],
             (16 * jnp.arange(8))[:, None, None] + jnp.arange(d)[None, None, :]
             ].set(jnp.broadcast_to(w2[None], (8, d, d)))
    f = f.astype(jnp.bfloat16)

    g_all = X_cat // vb
    s_all = (X_cat % vb) // sub
    r_all = X_cat % sub

    splits = [(0, 16), (16, n_cat)]
    vals = []
    for j0, j1 in splits:
        jn = j1 - j0
        t2 = _tc_transform(tables_t, f, j0, jn, vb=vb)
        t2_flat = t2.reshape(jn * gv * sub * 8, d)
        offs = (jnp.arange(jn, dtype=jnp.int32) * gv)[None, :]
        flat_idx = (
            (((offs + g_all[:, j0:j1]) * sub + r_all[:, j0:j1]) * 8)
            + s_all[:, j0:j1]
        ).reshape(-1)
        assert flat_idx.shape[0] % 128 == 0
        idx2d = flat_idx.reshape(-1, 128)
        rows = _sc_gather(t2_flat, idx2d)
        vals.append(rows.reshape(bsz, jn * d))

    w0 = vals[0].shape[1]
    w1c = vals[1].shape[1]
    m0 = jnp.pad(jnp.eye(w0, dtype=jnp.float32),
                 ((0, 0), ((1 + n_num) * d, wo - (1 + n_num) * d - w0)))
    m1 = jnp.pad(jnp.eye(w1c, dtype=jnp.float32),
                 ((0, 0), ((1 + n_num) * d + w0, 0)))

    out2d = _tc_fuse(X_num, vals[0], vals[1], m_num, m0, m1, c_row)
    return out2d.reshape(bsz, n_tok, d)
